# Initial kernel scaffold; baseline (speedup 1.0000x reference)
import jax
import jax.numpy as jnp
from jax.experimental import pallas as pl


def kernel(x, edge_index, ts, t, time_embedding, WQ, WK, WV):
    raise NotImplementedError("write your pallas kernel here")


# speedup vs baseline: 1.2392x; 1.2392x over previous
import functools

import jax
import jax.numpy as jnp
from jax import lax
from jax.experimental import pallas as pl
from jax.experimental.pallas import tpu as pltpu

MAXT = 10000.0
ROWS = 50000
DIM = 128
N = 10000
E = 320000
DFEAT = 128
OUT = 128
DELTAT = MAXT / ROWS


def _proj_body(phi_t_ref, wq_t_ref, x_ref, w_ref, o_ref):
    acc = jnp.dot(x_ref[...], w_ref[...], preferred_element_type=jnp.float32)
    qt = jnp.dot(phi_t_ref[...], wq_t_ref[...], preferred_element_type=jnp.float32)
    o_ref[...] = jnp.concatenate([acc[:, :OUT] + qt, acc[:, OUT:]], axis=1)


def _node_proj(x, w_all, phi_t, wq_t):
    blk = 2000
    return pl.pallas_call(
        _proj_body,
        grid=(N // blk,),
        in_specs=[
            pl.BlockSpec((1, DIM), lambda i: (0, 0)),
            pl.BlockSpec((DIM, OUT), lambda i: (0, 0)),
            pl.BlockSpec((blk, DFEAT), lambda i: (i, 0)),
            pl.BlockSpec((DFEAT, 3 * OUT), lambda i: (0, 0)),
        ],
        out_specs=pl.BlockSpec((blk, 3 * OUT), lambda i: (i, 0)),
        out_shape=jax.ShapeDtypeStruct((N, 3 * OUT), jnp.float32),
    )(phi_t, wq_t, x, w_all)


def _te_body(te_ref, w_ref, o_ref):
    o_ref[...] = jnp.dot(te_ref[...], w_ref[...], preferred_element_type=jnp.float32)


def _te_proj(te, w_kv):
    blk = 2000
    return pl.pallas_call(
        _te_body,
        grid=(ROWS // blk,),
        in_specs=[
            pl.BlockSpec((blk, DIM), lambda i: (i, 0)),
            pl.BlockSpec((DIM, 2 * OUT), lambda i: (0, 0)),
        ],
        out_specs=pl.BlockSpec((blk, 2 * OUT), lambda i: (i, 0)),
        out_shape=jax.ShapeDtypeStruct((ROWS, 2 * OUT), jnp.float32),
    )(te, w_kv)


def kernel(x, edge_index, ts, t, time_embedding, WQ, WK, WV):
    src = edge_index[0]
    dst = edge_index[1]
    idx_ts = jnp.clip(jnp.floor(ts / DELTAT), 0, ROWS - 1).astype(jnp.int32)
    idx_t = jnp.clip(jnp.floor(t / DELTAT), 0, ROWS - 1).astype(jnp.int32)
    phi_t = lax.dynamic_slice_in_dim(time_embedding, idx_t, 1, axis=0)

    w_all = jnp.concatenate([WQ[:DFEAT], WK[:DFEAT], WV[:DFEAT]], axis=1)
    w_kv = jnp.concatenate([WK[DFEAT:], WV[DFEAT:]], axis=1)

    node_proj = _node_proj(x, w_all, phi_t, WQ[DFEAT:])
    te_proj = _te_proj(time_embedding, w_kv)

    qn = node_proj[:, :OUT]
    xk = node_proj[:, OUT:2 * OUT]
    xv = node_proj[:, 2 * OUT:]
    tk = te_proj[:, :OUT]
    tv = te_proj[:, OUT:]

    q_e = jnp.take(qn, dst, axis=0)
    k_e = jnp.take(xk, src, axis=0) + jnp.take(tk, idx_ts, axis=0)
    v_e = jnp.take(xv, src, axis=0) + jnp.take(tv, idx_ts, axis=0)
    alpha = jnp.sum(q_e * k_e, axis=1)
    amax = jax.ops.segment_max(alpha, dst, num_segments=N)
    amax = jnp.where(jnp.isfinite(amax), amax, 0.0)
    ex = jnp.exp(alpha - jnp.take(amax, dst, axis=0))
    den = jax.ops.segment_sum(ex, dst, num_segments=N)
    num = jax.ops.segment_sum(v_e * ex[:, None], dst, num_segments=N)
    return num / jnp.maximum(den, 1e-16)[:, None]


# speedup vs baseline: 4.8455x; 3.9102x over previous
import functools

import jax
import jax.numpy as jnp
from jax import lax
from jax.experimental import pallas as pl
from jax.experimental.pallas import tpu as pltpu
from jax.experimental.pallas import tpu_sc as plsc

MAXT = 10000.0
ROWS = 50000
DIM = 128
N = 10000
E = 320000
DFEAT = 128
OUT = 128
DELTAT = MAXT / ROWS

NC = 2
NS = 16
NW = NC * NS
EPW = E // NW
C = 80
NCHUNK = EPW // C
NEG = -3.0e38


def _proj_body(phi_t_ref, wq_t_ref, x_ref, w_ref, q_ref, k_ref, v_ref):
    acc = jnp.dot(x_ref[...], w_ref[...], preferred_element_type=jnp.float32)
    qt = jnp.dot(phi_t_ref[...], wq_t_ref[...], preferred_element_type=jnp.float32)
    q_ref[...] = acc[:, :OUT] + qt
    k_ref[...] = acc[:, OUT:2 * OUT]
    v_ref[...] = acc[:, 2 * OUT:]


def _node_proj(x, w_all, phi_t, wq_t):
    blk = 2000
    return pl.pallas_call(
        _proj_body,
        grid=(N // blk,),
        in_specs=[
            pl.BlockSpec((1, DIM), lambda i: (0, 0)),
            pl.BlockSpec((DIM, OUT), lambda i: (0, 0)),
            pl.BlockSpec((blk, DFEAT), lambda i: (i, 0)),
            pl.BlockSpec((DFEAT, 3 * OUT), lambda i: (0, 0)),
        ],
        out_specs=[
            pl.BlockSpec((blk, OUT), lambda i: (i, 0)),
            pl.BlockSpec((blk, OUT), lambda i: (i, 0)),
            pl.BlockSpec((blk, OUT), lambda i: (i, 0)),
        ],
        out_shape=[jax.ShapeDtypeStruct((N, OUT), jnp.float32)] * 3,
    )(phi_t, wq_t, x, w_all)


def _te_body(te_ref, w_ref, k_ref, v_ref):
    acc = jnp.dot(te_ref[...], w_ref[...], preferred_element_type=jnp.float32)
    k_ref[...] = acc[:, :OUT]
    v_ref[...] = acc[:, OUT:]


def _te_proj(te, w_kv):
    blk = 2000
    return pl.pallas_call(
        _te_body,
        grid=(ROWS // blk,),
        in_specs=[
            pl.BlockSpec((blk, DIM), lambda i: (i, 0)),
            pl.BlockSpec((DIM, 2 * OUT), lambda i: (0, 0)),
        ],
        out_specs=[
            pl.BlockSpec((blk, OUT), lambda i: (i, 0)),
            pl.BlockSpec((blk, OUT), lambda i: (i, 0)),
        ],
        out_shape=[jax.ShapeDtypeStruct((ROWS, OUT), jnp.float32)] * 2,
    )(te, w_kv)


def _amax_combine_body(t_ref, o_ref):
    o_ref[...] = jnp.max(t_ref[...], axis=0)


def _amax_combine(tables):
    return pl.pallas_call(
        _amax_combine_body,
        out_shape=jax.ShapeDtypeStruct((N,), jnp.float32),
    )(tables)


def _finalize_body(num_ref, den_ref, o_ref):
    den = den_ref[0] + den_ref[1]
    num = num_ref[0] + num_ref[1]
    o_ref[...] = num / jnp.maximum(den, 1e-16)[:, None]


def _finalize(num, den):
    blk = 2048
    return pl.pallas_call(
        _finalize_body,
        grid=(NP // blk,),
        in_specs=[
            pl.BlockSpec((NC, blk, OUT), lambda i: (0, i, 0)),
            pl.BlockSpec((NC, blk), lambda i: (0, i)),
        ],
        out_specs=pl.BlockSpec((blk, OUT), lambda i: (i, 0)),
        out_shape=jax.ShapeDtypeStruct((NP, OUT), jnp.float32),
    )(num, den)


def _scatter_max(table_ref, d, a):
    def body(r, _):
        del r
        cur = plsc.load_gather(table_ref, [d])
        plsc.store_scatter(table_ref, [d], a, mask=a > cur)
        return 0

    lax.fori_loop(0, 16, body, 0)


def _pass1_kernel(qn, xk, tk, src, dst, its):
    mesh = plsc.VectorSubcoreMesh(core_axis_name="c", subcore_axis_name="s")

    @functools.partial(
        pl.kernel,
        out_type=[
            jax.ShapeDtypeStruct((E,), jnp.float32),
            jax.ShapeDtypeStruct((NW, N), jnp.float32),
        ],
        mesh=mesh,
        scratch_types=[
            pltpu.VMEM((C,), jnp.int32),
            pltpu.VMEM((C,), jnp.int32),
            pltpu.VMEM((C,), jnp.int32),
            pltpu.VMEM((C, OUT), jnp.float32),
            pltpu.VMEM((C, OUT), jnp.float32),
            pltpu.VMEM((C, OUT), jnp.float32),
            pltpu.VMEM((C,), jnp.float32),
            pltpu.VMEM((256,), jnp.float32),
            pltpu.VMEM((N,), jnp.float32),
            pltpu.SemaphoreType.DMA,
        ],
        compiler_params=pltpu.CompilerParams(needs_layout_passes=False),
    )
    def k(qn_h, xk_h, tk_h, src_h, dst_h, its_h, alpha_h, amax_h,
          srcb, dstb, itsb, bufq, bufk, buft, alphab, accb, amaxv, sem):
        c = lax.axis_index("c")
        s = lax.axis_index("s")
        w = c * NS + s
        base0 = w * EPW

        def init(i, _):
            amaxv[pl.ds(i * 16, 16)] = jnp.full((16,), NEG, jnp.float32)
            return 0
        lax.fori_loop(0, N // 16, init, 0)

        def chunk(i, _):
            base = base0 + i * C
            pltpu.sync_copy(src_h.at[pl.ds(base, C)], srcb)
            pltpu.sync_copy(dst_h.at[pl.ds(base, C)], dstb)
            pltpu.sync_copy(its_h.at[pl.ds(base, C)], itsb)
            cq = pltpu.async_copy(qn_h.at[dstb], bufq, sem)
            ck = pltpu.async_copy(xk_h.at[srcb], bufk, sem)
            ct = pltpu.async_copy(tk_h.at[itsb], buft, sem)
            cq.wait()
            ck.wait()
            ct.wait()

            def group(g, _):
                lane = lax.iota(jnp.int32, 16)

                def edge(j, _):
                    e = g * 16 + j
                    acc = jnp.zeros((16,), jnp.float32)
                    for d in range(8):
                        sl = pl.ds(d * 16, 16)
                        acc = acc + bufq[e, sl] * (bufk[e, sl] + buft[e, sl])
                    accb[pl.ds(j * 16, 16)] = acc
                    return 0
                lax.fori_loop(0, 16, edge, 0)

                av = jnp.zeros((16,), jnp.float32)
                for d in range(16):
                    av = av + plsc.load_gather(accb, [lane * 16 + d])
                alphab[pl.ds(g * 16, 16)] = av
                dd = dstb[pl.ds(g * 16, 16)]
                _scatter_max(amaxv, dd, av)
                return 0
            lax.fori_loop(0, C // 16, group, 0)
            pltpu.sync_copy(alphab, alpha_h.at[pl.ds(base, C)])
            return 0
        lax.fori_loop(0, NCHUNK, chunk, 0)
        pltpu.sync_copy(amaxv, amax_h.at[w])

    return k(qn, xk, tk, src, dst, its)


NP = 10240


def _pass2_kernel(xv, tv, src, dst, its, alpha, amax):
    mesh = plsc.VectorSubcoreMesh(core_axis_name="c", subcore_axis_name="s")
    ZR = 128

    @functools.partial(
        pl.kernel,
        out_type=[
            jax.ShapeDtypeStruct((NC, NP, OUT), jnp.float32),
            jax.ShapeDtypeStruct((NC, NP), jnp.float32),
        ],
        mesh=mesh,
        scratch_types=[
            pltpu.VMEM((C,), jnp.int32),
            pltpu.VMEM((C,), jnp.int32),
            pltpu.VMEM((C,), jnp.int32),
            pltpu.VMEM((C, OUT), jnp.float32),
            pltpu.VMEM((C, OUT), jnp.float32),
            pltpu.VMEM((C,), jnp.float32),
            pltpu.VMEM((C,), jnp.float32),
            pltpu.VMEM((N,), jnp.float32),
            pltpu.VMEM((ZR, OUT), jnp.float32),
            pltpu.VMEM((NP // NS,), jnp.float32),
            pltpu.VMEM_SHARED((NP, OUT), jnp.float32),
            pltpu.VMEM_SHARED((NP,), jnp.float32),
            pltpu.SemaphoreType.DMA,
        ],
        compiler_params=pltpu.CompilerParams(needs_layout_passes=False),
    )
    def k(xv_h, tv_h, src_h, dst_h, its_h, alpha_h, amax_h, num_h, den_h,
          srcb, dstb, itsb, bufa, bufb, alphab, exb, amaxv, zbuf, zbufd,
          numsp, densp, sem):
        c = lax.axis_index("c")
        s = lax.axis_index("s")
        w = c * NS + s
        base0 = w * EPW

        pltpu.sync_copy(amax_h, amaxv)

        z16 = jnp.zeros((16,), jnp.float32)
        rpt = NP // NS

        def initz(i, _):
            for d in range(OUT // 16):
                zbuf[i, pl.ds(d * 16, 16)] = z16
            return 0
        lax.fori_loop(0, ZR, initz, 0)

        def initzd(i, _):
            zbufd[pl.ds(i * 16, 16)] = z16
            return 0
        lax.fori_loop(0, rpt // 16, initzd, 0)

        for kk in range(rpt // ZR):
            pltpu.sync_copy(zbuf, numsp.at[pl.ds(s * rpt + kk * ZR, ZR)])
        pltpu.sync_copy(zbufd, densp.at[pl.ds(s * rpt, rpt)])
        plsc.subcore_barrier()

        def chunk(i, _):
            base = base0 + i * C
            pltpu.sync_copy(src_h.at[pl.ds(base, C)], srcb)
            pltpu.sync_copy(dst_h.at[pl.ds(base, C)], dstb)
            pltpu.sync_copy(its_h.at[pl.ds(base, C)], itsb)
            pltpu.sync_copy(alpha_h.at[pl.ds(base, C)], alphab)
            ca = pltpu.async_copy(xv_h.at[srcb], bufa, sem)
            cb = pltpu.async_copy(tv_h.at[itsb], bufb, sem)
            ca.wait()
            cb.wait()

            lane = lax.iota(jnp.int32, 16)

            def group(g, _):
                sl = pl.ds(g * 16, 16)
                dd = dstb[sl]
                a = alphab[sl]
                m = plsc.load_gather(amaxv, [dd])
                ex = jnp.exp(a - m)
                exb[sl] = ex

                def edge(j, _):
                    e = g * 16 + j
                    exv = plsc.load_gather(
                        exb, [jnp.full((16,), e, jnp.int32)])
                    for d in range(8):
                        dsl = pl.ds(d * 16, 16)
                        bufa[e, dsl] = (bufa[e, dsl] + bufb[e, dsl]) * exv
                    return 0
                lax.fori_loop(0, 16, edge, 0)
                return 0
            lax.fori_loop(0, C // 16, group, 0)
            pltpu.sync_copy(bufa, numsp.at[dstb], add=True)
            pltpu.sync_copy(exb, densp.at[dstb], add=True)
            return 0
        lax.fori_loop(0, NCHUNK, chunk, 0)

        plsc.subcore_barrier()
        pltpu.sync_copy(numsp.at[pl.ds(s * rpt, rpt)],
                        num_h.at[c, pl.ds(s * rpt, rpt)])
        pltpu.sync_copy(densp.at[pl.ds(s * rpt, rpt)],
                        den_h.at[c, pl.ds(s * rpt, rpt)])

    return k(xv, tv, src, dst, its, alpha, amax)


def kernel(x, edge_index, ts, t, time_embedding, WQ, WK, WV):
    src = edge_index[0]
    dst = edge_index[1]
    idx_ts = jnp.clip(jnp.floor(ts / DELTAT), 0, ROWS - 1).astype(jnp.int32)
    idx_t = jnp.clip(jnp.floor(t / DELTAT), 0, ROWS - 1).astype(jnp.int32)
    phi_t = lax.dynamic_slice_in_dim(time_embedding, idx_t, 1, axis=0)

    w_all = jnp.concatenate([WQ[:DFEAT], WK[:DFEAT], WV[:DFEAT]], axis=1)
    w_kv = jnp.concatenate([WK[DFEAT:], WV[DFEAT:]], axis=1)

    qn, xk, xv = _node_proj(x, w_all, phi_t, WQ[DFEAT:])
    tk, tv = _te_proj(time_embedding, w_kv)

    alpha, amax_tiles = _pass1_kernel(qn, xk, tk, src, dst, idx_ts)
    amax = _amax_combine(amax_tiles)
    num, den = _pass2_kernel(xv, tv, src, dst, idx_ts, alpha, amax)
    return _finalize(num, den)[:N]
